# baseline (device time: 90553 ns/iter reference)
import jax
import jax.numpy as jnp
from jax import lax
from jax.experimental import pallas as pl
from jax.experimental.pallas import tpu as pltpu

N_DEV = 4
SQ = 512
D = 1024
DH = 128
HQ_LOCAL = 8
GROUP = 4
SCALE = 0.08838834764831843


def kernel(x, Wq, Wo, Wk, Wv):
    my = lax.axis_index("i")
    Wk_loc = lax.dynamic_slice_in_dim(Wk, my * 2 * DH, 2 * DH, axis=1)
    Wv_loc = lax.dynamic_slice_in_dim(Wv, my * 2 * DH, 2 * DH, axis=1)
    x2 = x.reshape(SQ, D)

    def body(x_ref, wq_ref, wo_ref, wk_ref, wv_ref, out_ref,
             comm_ref, send_sems, recv_sems):
        my_pos = lax.axis_index("i")
        left = lax.rem(my_pos + N_DEV - 1, N_DEV)
        right = lax.rem(my_pos + 1, N_DEV)

        barrier_sem = pltpu.get_barrier_semaphore()
        for nbr in [left, right]:
            pl.semaphore_signal(
                barrier_sem, inc=1,
                device_id=(nbr,), device_id_type=pl.DeviceIdType.MESH,
            )
        pl.semaphore_wait(barrier_sem, 2)

        xv = x_ref[:, :]
        Q = jnp.dot(xv, wq_ref[:, :], preferred_element_type=jnp.float32)
        K = jnp.dot(xv, wk_ref[:, :], preferred_element_type=jnp.float32)
        V = jnp.dot(xv, wv_ref[:, :], preferred_element_type=jnp.float32)

        outs = []
        for j in range(HQ_LOCAL):
            qj = Q[:, j * DH:(j + 1) * DH]
            g = j // GROUP
            kg = K[:, g * DH:(g + 1) * DH]
            vg = V[:, g * DH:(g + 1) * DH]
            s = lax.dot_general(
                qj, kg, (((1,), (1,)), ((), ())),
                preferred_element_type=jnp.float32,
            ) * SCALE
            m = jnp.max(s, axis=1, keepdims=True)
            p = jnp.exp(s - m)
            l = jnp.sum(p, axis=1, keepdims=True)
            outs.append(jnp.dot(p, vg, preferred_element_type=jnp.float32) / l)
        attn = jnp.concatenate(outs, axis=1)
        partial = jnp.dot(attn, wo_ref[:, :], preferred_element_type=jnp.float32)

        comm_ref[0] = partial
        acc = partial
        for h in range(N_DEV - 1):
            rdma = pltpu.make_async_remote_copy(
                src_ref=comm_ref.at[h],
                dst_ref=comm_ref.at[h + 1],
                send_sem=send_sems.at[h],
                recv_sem=recv_sems.at[h],
                device_id=(right,),
                device_id_type=pl.DeviceIdType.MESH,
            )
            rdma.start()
            rdma.wait()
            acc = acc + comm_ref[h + 1]
        out_ref[:, :] = acc

    out = pl.pallas_call(
        body,
        out_shape=jax.ShapeDtypeStruct((SQ, D), jnp.float32),
        in_specs=[pl.BlockSpec(memory_space=pltpu.VMEM)] * 5,
        out_specs=pl.BlockSpec(memory_space=pltpu.VMEM),
        scratch_shapes=[
            pltpu.VMEM((N_DEV, SQ, D), jnp.float32),
            pltpu.SemaphoreType.DMA((N_DEV - 1,)),
            pltpu.SemaphoreType.DMA((N_DEV - 1,)),
        ],
        compiler_params=pltpu.CompilerParams(collective_id=0),
    )(x2, Wq, Wo, Wk_loc, Wv_loc)
    return out.reshape(1, SQ, D)


# device time: 32886 ns/iter; 2.7535x vs baseline; 2.7535x over previous
import jax
import jax.numpy as jnp
from jax import lax
from jax.experimental import pallas as pl
from jax.experimental.pallas import tpu as pltpu

N_DEV = 4
SQ = 512
D = 1024
DH = 128
HQ_LOCAL = 8
GROUP = 4
CHUNK = SQ // N_DEV
SCALE = 0.08838834764831843


def kernel(x, Wq, Wo, Wk, Wv):
    my = lax.axis_index("i")
    Wk_loc = lax.dynamic_slice_in_dim(Wk, my * 2 * DH, 2 * DH, axis=1)
    Wv_loc = lax.dynamic_slice_in_dim(Wv, my * 2 * DH, 2 * DH, axis=1)
    x2 = x.reshape(SQ, D)

    def body(x_ref, wq_ref, wo_ref, wk_ref, wv_ref, out_ref,
             send_ref, rs_recv_ref, ag_ref,
             rs_send_sems, rs_recv_sems, ag_send_sems, ag_recv_sems):
        p = lax.axis_index("i")

        barrier_sem = pltpu.get_barrier_semaphore()
        for k in range(1, N_DEV):
            pl.semaphore_signal(
                barrier_sem, inc=1,
                device_id=(lax.rem(p + k, N_DEV),),
                device_id_type=pl.DeviceIdType.MESH,
            )
        pl.semaphore_wait(barrier_sem, N_DEV - 1)

        xv = x_ref[:, :]
        Q = jnp.dot(xv, wq_ref[:, :], preferred_element_type=jnp.float32)
        K = jnp.dot(xv, wk_ref[:, :], preferred_element_type=jnp.float32)
        V = jnp.dot(xv, wv_ref[:, :], preferred_element_type=jnp.float32)

        outs = []
        for j in range(HQ_LOCAL):
            qj = Q[:, j * DH:(j + 1) * DH]
            g = j // GROUP
            kg = K[:, g * DH:(g + 1) * DH]
            vg = V[:, g * DH:(g + 1) * DH]
            s = lax.dot_general(
                qj, kg, (((1,), (1,)), ((), ())),
                preferred_element_type=jnp.float32,
            ) * SCALE
            m = jnp.max(s, axis=1, keepdims=True)
            pj = jnp.exp(s - m)
            l = jnp.sum(pj, axis=1, keepdims=True)
            outs.append(jnp.dot(pj, vg, preferred_element_type=jnp.float32) / l)
        attn = jnp.concatenate(outs, axis=1)
        partial = jnp.dot(attn, wo_ref[:, :], preferred_element_type=jnp.float32)

        send_ref[...] = partial.astype(jnp.bfloat16).reshape(N_DEV, CHUNK, D)

        rs = []
        for k in range(1, N_DEV):
            t = lax.rem(p + k, N_DEV)
            r = pltpu.make_async_remote_copy(
                src_ref=send_ref.at[t],
                dst_ref=rs_recv_ref.at[N_DEV - 1 - k],
                send_sem=rs_send_sems.at[k - 1],
                recv_sem=rs_recv_sems.at[N_DEV - 1 - k],
                device_id=(t,),
                device_id_type=pl.DeviceIdType.MESH,
            )
            r.start()
            rs.append(r)
        for r in rs:
            r.wait_recv()

        acc = send_ref[p].astype(jnp.float32)
        for slot in range(N_DEV - 1):
            acc = acc + rs_recv_ref[slot].astype(jnp.float32)
        out_ref[p] = acc
        ag_ref[p] = acc.astype(jnp.bfloat16)

        ag = []
        for k in range(1, N_DEV):
            t = lax.rem(p + k, N_DEV)
            a = pltpu.make_async_remote_copy(
                src_ref=ag_ref.at[p],
                dst_ref=ag_ref.at[p],
                send_sem=ag_send_sems.at[k - 1],
                recv_sem=ag_recv_sems.at[N_DEV - 1 - k],
                device_id=(t,),
                device_id_type=pl.DeviceIdType.MESH,
            )
            a.start()
            ag.append(a)
        for r in rs:
            r.wait_send()
        for a in ag:
            a.wait_recv()
        for k in range(1, N_DEV):
            t = lax.rem(p + k, N_DEV)
            out_ref[t] = ag_ref[t].astype(jnp.float32)
        for a in ag:
            a.wait_send()

    out = pl.pallas_call(
        body,
        out_shape=jax.ShapeDtypeStruct((N_DEV, CHUNK, D), jnp.float32),
        in_specs=[pl.BlockSpec(memory_space=pltpu.VMEM)] * 5,
        out_specs=pl.BlockSpec(memory_space=pltpu.VMEM),
        scratch_shapes=[
            pltpu.VMEM((N_DEV, CHUNK, D), jnp.bfloat16),
            pltpu.VMEM((N_DEV - 1, CHUNK, D), jnp.bfloat16),
            pltpu.VMEM((N_DEV, CHUNK, D), jnp.bfloat16),
            pltpu.SemaphoreType.DMA((N_DEV - 1,)),
            pltpu.SemaphoreType.DMA((N_DEV - 1,)),
            pltpu.SemaphoreType.DMA((N_DEV - 1,)),
            pltpu.SemaphoreType.DMA((N_DEV - 1,)),
        ],
        compiler_params=pltpu.CompilerParams(collective_id=0),
    )(x2, Wq, Wo, Wk_loc, Wv_loc)
    return out.reshape(1, SQ, D)
